# baseline (speedup 1.0000x reference)
import functools

import jax
import jax.numpy as jnp
from jax.experimental import pallas as pl
from jax.experimental.pallas import tpu as pltpu

_NUM_EXPERTS = 16
_TOP_K = 2
_BLK = 512


def _gate_kernel(x_ref, w_ref, o_ref):
    lt = jax.lax.dot_general(
        w_ref[...], x_ref[...], (((1,), (1,)), ((), ())),
        preferred_element_type=jnp.float32,
    )
    m = jnp.max(lt, axis=0, keepdims=True)
    row = jax.lax.broadcasted_iota(jnp.int32, lt.shape, 0)
    i1 = jnp.min(jnp.where(lt == m, row, _NUM_EXPERTS), axis=0, keepdims=True)
    masked = jnp.where(row == i1, -jnp.inf, lt)
    l2 = jnp.max(masked, axis=0, keepdims=True)
    i2 = jnp.min(jnp.where(masked == l2, row, _NUM_EXPERTS), axis=0, keepdims=True)
    z = jnp.sum(jnp.exp(lt - m), axis=0, keepdims=True)

    v1 = 1.0 / z
    t = jnp.exp(jnp.exp(l2 - m) / z - v1)
    d = 1.0 + t
    p1 = 1.0 / d
    p2 = t / d
    b1 = jax.lax.bitcast_convert_type(i1, jnp.float32)
    b2 = jax.lax.bitcast_convert_type(i2, jnp.float32)
    o_ref[...] = jnp.concatenate([p1, p2, b1, b2, p1, p1, p1, p1], axis=0)


@functools.partial(jax.jit, static_argnums=())
def kernel(x, W_gate):
    b, s, h = x.shape
    rows = b * s
    x_flat = x.reshape(rows, h)
    grid = (rows // _BLK,)
    packed = pl.pallas_call(
        _gate_kernel,
        grid=grid,
        in_specs=[
            pl.BlockSpec((_BLK, h), lambda i: (i, 0)),
            pl.BlockSpec((_NUM_EXPERTS, h), lambda i: (0, 0)),
        ],
        out_specs=pl.BlockSpec((8, _BLK), lambda i: (0, i)),
        out_shape=jax.ShapeDtypeStruct((8, rows), jnp.float32),
        compiler_params=pltpu.CompilerParams(
            dimension_semantics=("arbitrary",),
        ),
    )(x_flat, W_gate)
    scores = packed[0:2].T
    idx = jax.lax.bitcast_convert_type(packed[2:4], jnp.int32).T
    aux_loss = jnp.array(0.0, dtype=jnp.float32)
    return (scores, idx, aux_loss)


# speedup vs baseline: 1.1300x; 1.1300x over previous
import functools

import jax
import jax.numpy as jnp
from jax.experimental import pallas as pl
from jax.experimental.pallas import tpu as pltpu

_NUM_EXPERTS = 16
_TOP_K = 2
_BLK = 1024


def _gate_kernel(x_ref, w_ref, o_ref):
    lt = jax.lax.dot_general(
        w_ref[...], x_ref[...], (((1,), (1,)), ((), ())),
        preferred_element_type=jnp.float32,
    )
    m = jnp.max(lt, axis=0, keepdims=True)
    row = jax.lax.broadcasted_iota(jnp.int32, lt.shape, 0)
    i1 = jnp.min(jnp.where(lt == m, row, _NUM_EXPERTS), axis=0, keepdims=True)
    masked = jnp.where(row == i1, -jnp.inf, lt)
    l2 = jnp.max(masked, axis=0, keepdims=True)
    i2 = jnp.min(jnp.where(masked == l2, row, _NUM_EXPERTS), axis=0, keepdims=True)
    z = jnp.sum(jnp.exp(lt - m), axis=0, keepdims=True)

    v1 = 1.0 / z
    t = jnp.exp(jnp.exp(l2 - m) / z - v1)
    d = 1.0 + t
    p1 = 1.0 / d
    p2 = t / d
    b1 = jax.lax.bitcast_convert_type(i1, jnp.float32)
    b2 = jax.lax.bitcast_convert_type(i2, jnp.float32)
    o_ref[...] = jnp.concatenate([p1, p2, b1, b2, p1, p1, p1, p1], axis=0)


@functools.partial(jax.jit, static_argnums=())
def kernel(x, W_gate):
    b, s, h = x.shape
    rows = b * s
    x_flat = x.reshape(rows, h)
    grid = (rows // _BLK,)
    packed = pl.pallas_call(
        _gate_kernel,
        grid=grid,
        in_specs=[
            pl.BlockSpec((_BLK, h), lambda i: (i, 0)),
            pl.BlockSpec((_NUM_EXPERTS, h), lambda i: (0, 0)),
        ],
        out_specs=pl.BlockSpec((8, _BLK), lambda i: (0, i)),
        out_shape=jax.ShapeDtypeStruct((8, rows), jnp.float32),
        compiler_params=pltpu.CompilerParams(
            dimension_semantics=("arbitrary",),
        ),
    )(x_flat, W_gate)
    pt = packed.T
    scores = pt[:, 0:2]
    idx = jax.lax.bitcast_convert_type(pt[:, 2:4], jnp.int32)
    aux_loss = jnp.array(0.0, dtype=jnp.float32)
    return (scores, idx, aux_loss)
